# Initial kernel scaffold; baseline (speedup 1.0000x reference)
import jax
import jax.numpy as jnp
from jax.experimental import pallas as pl


def kernel(x, graph_level_feats, edge_attr, edge_index, batch_index, convs, lins):
    raise NotImplementedError("write your pallas kernel here")


# speedup vs baseline: 1.8589x; 1.8589x over previous
import functools

import jax
import jax.numpy as jnp
from jax import lax
from jax.experimental import pallas as pl
from jax.experimental.pallas import tpu as pltpu
from jax.experimental.pallas import tpu_sc as plsc


def _pick_chunk(epw):
    k = 128 - 128 % 8
    while k >= 8:
        if epw % k == 0 and k % 8 == 0:
            return k
        k -= 8
    raise ValueError(epw)


def _sc_message_pass(Dc, N, E):
    NS = 16
    EPW = E // NS
    K = _pick_chunk(EPW)
    NCH = EPW // K
    RPW = (N // NS) // 8 * 8
    TAIL = N - NS * RPW
    mesh = plsc.VectorSubcoreMesh(core_axis_name="c", subcore_axis_name="s")

    @functools.partial(
        pl.kernel,
        mesh=mesh,
        out_type=[jax.ShapeDtypeStruct((N, Dc), jnp.float32),
                  jax.ShapeDtypeStruct((N, Dc), jnp.float32)],
        scratch_types=[
            pltpu.VMEM((K,), jnp.int32),
            pltpu.VMEM((K,), jnp.int32),
            pltpu.VMEM((K, Dc), jnp.float32),
            pltpu.VMEM((K, Dc), jnp.float32),
            pltpu.VMEM_SHARED((N, Dc), jnp.float32),
            pltpu.SemaphoreType.DMA,
        ],
    )
    def sc_fn(x_lo, x_hi, ew_lo, ew_hi, src, dst, out_lo, out_hi,
              srcv, dstv, mv, xv, acc, sem):
        s = lax.axis_index("s")
        c = lax.axis_index("c")
        r0 = s * RPW
        ebase = s * EPW

        def run(x_ref, ew_ref, out_ref):
            pltpu.sync_copy(x_ref.at[pl.ds(r0, RPW)], acc.at[pl.ds(r0, RPW)])

            @pl.when(s == NS - 1)
            def _():
                pltpu.sync_copy(x_ref.at[pl.ds(NS * RPW, TAIL)],
                                acc.at[pl.ds(NS * RPW, TAIL)])

            plsc.subcore_barrier()

            def chunk(j, carry):
                base = ebase + j * K
                pltpu.sync_copy(src.at[pl.ds(base, K)], srcv)
                pltpu.sync_copy(dst.at[pl.ds(base, K)], dstv)
                pltpu.sync_copy(ew_ref.at[pl.ds(base, K)], mv)
                pltpu.async_copy(x_ref.at[srcv], xv, sem).wait()

                def row(k, c2):
                    for gi in range(Dc // 16):
                        sl = pl.ds(gi * 16, 16)
                        mv[k, sl] = jnp.maximum(mv[k, sl] + xv[k, sl], 0.0)
                    return c2

                lax.fori_loop(0, K, row, 0)
                pltpu.sync_copy(mv, acc.at[dstv], add=True)
                return carry

            lax.fori_loop(0, NCH, chunk, 0)
            plsc.subcore_barrier()
            pltpu.sync_copy(acc.at[pl.ds(r0, RPW)], out_ref.at[pl.ds(r0, RPW)])

            @pl.when(s == NS - 1)
            def _():
                pltpu.sync_copy(acc.at[pl.ds(NS * RPW, TAIL)],
                                out_ref.at[pl.ds(NS * RPW, TAIL)])

        @pl.when(c == 0)
        def _():
            run(x_lo, ew_lo, out_lo)

        @pl.when(c == 1)
        def _():
            run(x_hi, ew_hi, out_hi)

    return sc_fn


def _sc_message_pass_esplit(D, N, E):
    NS = 16
    NW = 2 * NS
    EPW = E // NW
    K = _pick_chunk(EPW)
    NCH = EPW // K
    RPW = (N // NS) // 8 * 8
    TAIL = N - NS * RPW
    mesh = plsc.VectorSubcoreMesh(core_axis_name="c", subcore_axis_name="s")

    @functools.partial(
        pl.kernel,
        mesh=mesh,
        out_type=[jax.ShapeDtypeStruct((N, D), jnp.float32),
                  jax.ShapeDtypeStruct((N, D), jnp.float32)],
        scratch_types=[
            pltpu.VMEM((K,), jnp.int32),
            pltpu.VMEM((K,), jnp.int32),
            pltpu.VMEM((K, D), jnp.float32),
            pltpu.VMEM((K, D), jnp.float32),
            pltpu.VMEM_SHARED((N, D), jnp.float32),
            pltpu.SemaphoreType.DMA,
        ],
    )
    def sc_fn(x, ew, src, dst, out0, out1,
              srcv, dstv, mv, xv, acc, sem):
        s = lax.axis_index("s")
        c = lax.axis_index("c")
        r0 = s * RPW
        ebase = (c * NS + s) * EPW

        pltpu.sync_copy(x.at[pl.ds(r0, RPW)], acc.at[pl.ds(r0, RPW)])

        @pl.when(s == NS - 1)
        def _():
            pltpu.sync_copy(x.at[pl.ds(NS * RPW, TAIL)],
                            acc.at[pl.ds(NS * RPW, TAIL)])

        plsc.subcore_barrier()

        def chunk(j, carry):
            base = ebase + j * K
            pltpu.sync_copy(src.at[pl.ds(base, K)], srcv)
            pltpu.sync_copy(dst.at[pl.ds(base, K)], dstv)
            pltpu.sync_copy(ew.at[pl.ds(base, K)], mv)
            pltpu.async_copy(x.at[srcv], xv, sem).wait()

            def row(k, c2):
                for gi in range(D // 16):
                    sl = pl.ds(gi * 16, 16)
                    mv[k, sl] = jnp.maximum(mv[k, sl] + xv[k, sl], 0.0)
                return c2

            lax.fori_loop(0, K, row, 0)
            pltpu.sync_copy(mv, acc.at[dstv], add=True)
            return carry

        lax.fori_loop(0, NCH, chunk, 0)
        plsc.subcore_barrier()

        def writeout(out_ref):
            pltpu.sync_copy(acc.at[pl.ds(r0, RPW)], out_ref.at[pl.ds(r0, RPW)])

            @pl.when(s == NS - 1)
            def _():
                pltpu.sync_copy(acc.at[pl.ds(NS * RPW, TAIL)],
                                out_ref.at[pl.ds(NS * RPW, TAIL)])

        @pl.when(c == 0)
        def _():
            writeout(out0)

        @pl.when(c == 1)
        def _():
            writeout(out1)

    return sc_fn


def _edge_weights(edge_attr, Wes, bes, Ds):
    E, DE = edge_attr.shape
    BE = 2000
    grid = E // BE

    def body(ea_ref, *refs):
        w_refs = refs[:4]
        b_refs = refs[4:8]
        o_refs = refs[8:]
        ea = ea_ref[...]
        oi = 0
        for l in range(4):
            t = jnp.dot(ea, w_refs[l][...],
                        preferred_element_type=jnp.float32) + b_refs[l][...]
            if l == 0:
                o_refs[oi][...] = t
                oi += 1
            else:
                Dc = Ds[l] // 2
                o_refs[oi][...] = t[:, :Dc]
                o_refs[oi + 1][...] = t[:, Dc:]
                oi += 2

    in_specs = [pl.BlockSpec((BE, DE), lambda i: (i, 0))]
    in_specs += [pl.BlockSpec((DE, Ds[l]), lambda i: (0, 0)) for l in range(4)]
    in_specs += [pl.BlockSpec((1, Ds[l]), lambda i: (0, 0)) for l in range(4)]
    out_specs = [pl.BlockSpec((BE, Ds[0]), lambda i: (i, 0))]
    out_shapes = [jax.ShapeDtypeStruct((E, Ds[0]), jnp.float32)]
    for l in range(1, 4):
        Dc = Ds[l] // 2
        for _ in range(2):
            out_specs.append(pl.BlockSpec((BE, Dc), lambda i: (i, 0)))
            out_shapes.append(jax.ShapeDtypeStruct((E, Dc), jnp.float32))
    outs = pl.pallas_call(
        body, grid=(grid,), in_specs=in_specs, out_specs=out_specs,
        out_shape=out_shapes,
    )(edge_attr, *Wes, *[b.reshape(1, -1) for b in bes])
    return [outs[0], (outs[1], outs[2]), (outs[3], outs[4]), (outs[5], outs[6])]


def _mm_stats(h_lo, h_hi, W1, b1, xsub=None):
    N, Dc = h_lo.shape
    D, DH = W1.shape
    BN_ = 1000
    grid = N // BN_

    def body(lo_ref, hi_ref, *refs):
        if xsub is None:
            w_ref, b_ref, t_ref, st_ref = refs
            w = w_ref[...]
            t = (jnp.dot(lo_ref[...], w[:Dc, :],
                         preferred_element_type=jnp.float32)
                 + jnp.dot(hi_ref[...], w[Dc:, :],
                           preferred_element_type=jnp.float32)
                 + b_ref[...])
        else:
            x_ref, w_ref, b_ref, t_ref, st_ref = refs
            hp = lo_ref[...] + hi_ref[...] - x_ref[...]
            t = jnp.dot(hp, w_ref[...],
                        preferred_element_type=jnp.float32) + b_ref[...]
        t_ref[...] = t
        st = jnp.concatenate(
            [jnp.sum(t, axis=0, keepdims=True),
             jnp.sum(t * t, axis=0, keepdims=True)], axis=0)

        @pl.when(pl.program_id(0) == 0)
        def _():
            st_ref[...] = st

        @pl.when(pl.program_id(0) > 0)
        def _():
            st_ref[...] += st

    in_specs = [
        pl.BlockSpec((BN_, Dc), lambda i: (i, 0)),
        pl.BlockSpec((BN_, Dc), lambda i: (i, 0)),
    ]
    args = [h_lo, h_hi]
    if xsub is not None:
        in_specs.append(pl.BlockSpec((BN_, D), lambda i: (i, 0)))
        args.append(xsub)
    in_specs += [
        pl.BlockSpec((D, DH), lambda i: (0, 0)),
        pl.BlockSpec((1, DH), lambda i: (0, 0)),
    ]
    return pl.pallas_call(
        body, grid=(grid,),
        in_specs=in_specs,
        out_specs=[
            pl.BlockSpec((BN_, DH), lambda i: (i, 0)),
            pl.BlockSpec((2, DH), lambda i: (0, 0)),
        ],
        out_shape=[
            jax.ShapeDtypeStruct((N, DH), jnp.float32),
            jax.ShapeDtypeStruct((2, DH), jnp.float32),
        ],
    )(*args, W1, b1.reshape(1, -1))


def _bn_mm_pool(t, stats, g, bb, W2, b2, bi3, G, want_halves):
    N, DH = t.shape
    BN_ = 1000
    grid = N // BN_
    gb = jnp.stack([g, bb])
    inv_n = 1.0 / N

    def body(t_ref, st_ref, gb_ref, w_ref, b_ref, bi_ref, *o_refs):
        st = st_ref[...]
        mu = st[0:1, :] * inv_n
        var = st[1:2, :] * inv_n - (st[0:1, :] * inv_n) ** 2
        inv = lax.rsqrt(var + 1e-5)
        hn = jnp.maximum(gb_ref[0:1, :] * (t_ref[...] - mu) * inv
                         + gb_ref[1:2, :], 0.0)
        h = jnp.maximum(jnp.dot(hn, w_ref[...],
                                preferred_element_type=jnp.float32)
                        + b_ref[...], 0.0)
        if want_halves:
            o_refs[0][...] = h[:, :DH // 2]
            o_refs[1][...] = h[:, DH // 2:]
        p_ref = o_refs[-1]
        oh = (lax.broadcasted_iota(jnp.int32, (G, BN_), 0)
              == bi_ref[0]).astype(jnp.float32)
        pp = jnp.dot(oh, h, preferred_element_type=jnp.float32)

        @pl.when(pl.program_id(0) == 0)
        def _():
            p_ref[...] = pp

        @pl.when(pl.program_id(0) > 0)
        def _():
            p_ref[...] += pp

    out_specs = []
    out_shapes = []
    if want_halves:
        out_specs += [pl.BlockSpec((BN_, DH // 2), lambda i: (i, 0))] * 2
        out_shapes += [jax.ShapeDtypeStruct((N, DH // 2), jnp.float32)] * 2
    out_specs.append(pl.BlockSpec((G, DH), lambda i: (0, 0)))
    out_shapes.append(jax.ShapeDtypeStruct((G, DH), jnp.float32))
    return pl.pallas_call(
        body, grid=(grid,),
        in_specs=[
            pl.BlockSpec((BN_, DH), lambda i: (i, 0)),
            pl.BlockSpec((2, DH), lambda i: (0, 0)),
            pl.BlockSpec((2, DH), lambda i: (0, 0)),
            pl.BlockSpec((DH, DH), lambda i: (0, 0)),
            pl.BlockSpec((1, DH), lambda i: (0, 0)),
            pl.BlockSpec((1, 1, BN_), lambda i: (i, 0, 0)),
        ],
        out_specs=out_specs,
        out_shape=out_shapes,
    )(t, stats, gb, W2, b2.reshape(1, -1), bi3)


def _head_mm(xin, W, b, nblk, act):
    M, Kd = xin.shape
    _, Nd = W.shape
    BNN = Nd // nblk

    def body(x_ref, w_ref, b_ref, o_ref):
        v = jnp.dot(x_ref[...], w_ref[...],
                    preferred_element_type=jnp.float32) + b_ref[...]
        if act == "relu":
            v = jnp.maximum(v, 0.0)
        else:
            v = jnp.maximum(v, 0.0) + jnp.log(1.0 + jnp.exp(-jnp.abs(v)))
        o_ref[...] = v

    return pl.pallas_call(
        body, grid=(nblk,),
        in_specs=[
            pl.BlockSpec((M, Kd), lambda j: (0, 0)),
            pl.BlockSpec((Kd, BNN), lambda j: (0, j)),
            pl.BlockSpec((1, BNN), lambda j: (0, j)),
        ],
        out_specs=pl.BlockSpec((M, BNN), lambda j: (0, j)),
        out_shape=jax.ShapeDtypeStruct((M, Nd), jnp.float32),
    )(xin, W, b.reshape(1, -1))


def kernel(x, graph_level_feats, edge_attr, edge_index, batch_index, convs, lins):
    N, DF = x.shape
    E = edge_attr.shape[0]
    G = graph_level_feats.shape[0]
    DH = convs[0][4].shape[0]
    src = edge_index[0]
    dst = edge_index[1]
    Ds = [DF, DH, DH, DH]

    ews = _edge_weights(edge_attr, [c[0] for c in convs], [c[1] for c in convs], Ds)
    bi3 = batch_index.reshape(N // 1000, 1, 1000)

    h_lo, h_hi = None, None
    ps = []
    for l in range(4):
        We, be, W1, b1, g, bb, W2, b2 = convs[l]
        if l == 0:
            sc = _sc_message_pass_esplit(DF, N, E)
            a0, a1 = sc(x, ews[0], src, dst)
            t, stats = _mm_stats(a0, a1, W1, b1, xsub=x)
        else:
            sc = _sc_message_pass(Ds[l] // 2, N, E)
            hp_lo, hp_hi = sc(h_lo, h_hi, ews[l][0], ews[l][1], src, dst)
            t, stats = _mm_stats(hp_lo, hp_hi, W1, b1)
        res = _bn_mm_pool(t, stats, g, bb, W2, b2, bi3, G,
                          want_halves=(l < 3))
        if l < 3:
            h_lo, h_hi, p = res
        else:
            (p,) = res
        ps.append(p)

    skip = jnp.reshape(graph_level_feats, (G, -1)).astype(jnp.float32)
    cat = jnp.concatenate(ps + [skip], axis=1)
    Wa, ba, Wc, bc, Wb, bb2 = lins
    z = _head_mm(cat, Wa, ba, 16, "relu")
    z = _head_mm(z, Wc, bc, 16, "relu")
    return _head_mm(z, Wb, bb2, 1, "softplus")


# speedup vs baseline: 1.9874x; 1.0691x over previous
import functools

import jax
import jax.numpy as jnp
from jax import lax
from jax.experimental import pallas as pl
from jax.experimental.pallas import tpu as pltpu
from jax.experimental.pallas import tpu_sc as plsc


def _pick_chunk(epw):
    k = 128 - 128 % 8
    while k >= 8:
        if epw % k == 0 and k % 8 == 0 and (epw // k) % 2 == 1:
            return k
        k -= 8
    raise ValueError(epw)


def _edge_pipeline(x_ref, ew_ref, src, dst, acc, bufs, ebase, NCH, K, Dc):
    def load(cbase, b, first):
        srcv, dstv, mv, xv, s_src, s_dst, s_ew, s_g, s_sc = b
        if not first:
            pltpu.make_async_copy(mv, acc.at[dstv], s_sc).wait()
        pltpu.async_copy(src.at[pl.ds(cbase, K)], srcv, s_src)
        pltpu.async_copy(dst.at[pl.ds(cbase, K)], dstv, s_dst)
        pltpu.async_copy(ew_ref.at[pl.ds(cbase, K)], mv, s_ew)
        pltpu.make_async_copy(src.at[pl.ds(cbase, K)], srcv, s_src).wait()
        pltpu.async_copy(x_ref.at[srcv], xv, s_g)

    def compute(b):
        srcv, dstv, mv, xv, s_src, s_dst, s_ew, s_g, s_sc = b
        pltpu.make_async_copy(ew_ref.at[pl.ds(0, K)], mv, s_ew).wait()
        pltpu.make_async_copy(x_ref.at[srcv], xv, s_g).wait()
        pltpu.make_async_copy(dst.at[pl.ds(0, K)], dstv, s_dst).wait()

        def row(k, c2):
            for gi in range(Dc // 16):
                sl = pl.ds(gi * 16, 16)
                mv[k, sl] = jnp.maximum(mv[k, sl] + xv[k, sl], 0.0)
            return c2

        lax.fori_loop(0, K, row, 0, unroll=4)
        pltpu.async_copy(mv, acc.at[dstv], s_sc, add=True)

    A, B = bufs
    load(ebase, A, True)
    load(ebase + K, B, True)

    def body(i, carry):
        c0 = ebase + 2 * i * K
        compute(A)
        load(c0 + 2 * K, A, False)
        compute(B)
        load(c0 + 3 * K, B, False)
        return carry

    lax.fori_loop(0, (NCH - 3) // 2, body, 0)
    compute(A)
    load(ebase + (NCH - 1) * K, A, False)
    compute(B)
    compute(A)
    pltpu.make_async_copy(A[2], acc.at[A[1]], A[8]).wait()
    pltpu.make_async_copy(B[2], acc.at[B[1]], B[8]).wait()


def _sc_message_pass(Dc, N, E):
    NS = 16
    EPW = E // NS
    K = _pick_chunk(EPW)
    NCH = EPW // K
    RPW = (N // NS) // 8 * 8
    TAIL = N - NS * RPW
    mesh = plsc.VectorSubcoreMesh(core_axis_name="c", subcore_axis_name="s")

    @functools.partial(
        pl.kernel,
        mesh=mesh,
        out_type=[jax.ShapeDtypeStruct((N, Dc), jnp.float32),
                  jax.ShapeDtypeStruct((N, Dc), jnp.float32)],
        scratch_types=(
            [pltpu.VMEM((K,), jnp.int32)] * 2
            + [pltpu.VMEM((K,), jnp.int32)] * 2
            + [pltpu.VMEM((K, Dc), jnp.float32)] * 4
            + [pltpu.VMEM_SHARED((N, Dc), jnp.float32)]
            + [pltpu.SemaphoreType.DMA] * 10
        ),
    )
    def sc_fn(x_lo, x_hi, ew_lo, ew_hi, src, dst, out_lo, out_hi,
              srcva, srcvb, dstva, dstvb, mva, mvb, xva, xvb, acc, *sems):
        s = lax.axis_index("s")
        c = lax.axis_index("c")
        r0 = s * RPW
        ebase = s * EPW
        bufA = (srcva, dstva, mva, xva) + tuple(sems[:5])
        bufB = (srcvb, dstvb, mvb, xvb) + tuple(sems[5:])

        def run(x_ref, ew_ref, out_ref):
            pltpu.sync_copy(x_ref.at[pl.ds(r0, RPW)], acc.at[pl.ds(r0, RPW)])

            @pl.when(s == NS - 1)
            def _():
                pltpu.sync_copy(x_ref.at[pl.ds(NS * RPW, TAIL)],
                                acc.at[pl.ds(NS * RPW, TAIL)])

            plsc.subcore_barrier()
            _edge_pipeline(x_ref, ew_ref, src, dst, acc, (bufA, bufB),
                           ebase, NCH, K, Dc)
            plsc.subcore_barrier()
            pltpu.sync_copy(acc.at[pl.ds(r0, RPW)], out_ref.at[pl.ds(r0, RPW)])

            @pl.when(s == NS - 1)
            def _():
                pltpu.sync_copy(acc.at[pl.ds(NS * RPW, TAIL)],
                                out_ref.at[pl.ds(NS * RPW, TAIL)])

        @pl.when(c == 0)
        def _():
            run(x_lo, ew_lo, out_lo)

        @pl.when(c == 1)
        def _():
            run(x_hi, ew_hi, out_hi)

    return sc_fn


def _sc_message_pass_esplit(D, N, E):
    NS = 16
    NW = 2 * NS
    EPW = E // NW
    K = _pick_chunk(EPW)
    NCH = EPW // K
    RPW = (N // NS) // 8 * 8
    TAIL = N - NS * RPW
    mesh = plsc.VectorSubcoreMesh(core_axis_name="c", subcore_axis_name="s")

    @functools.partial(
        pl.kernel,
        mesh=mesh,
        out_type=[jax.ShapeDtypeStruct((N, D), jnp.float32),
                  jax.ShapeDtypeStruct((N, D), jnp.float32)],
        scratch_types=(
            [pltpu.VMEM((K,), jnp.int32)] * 2
            + [pltpu.VMEM((K,), jnp.int32)] * 2
            + [pltpu.VMEM((K, D), jnp.float32)] * 4
            + [pltpu.VMEM_SHARED((N, D), jnp.float32)]
            + [pltpu.SemaphoreType.DMA] * 10
        ),
    )
    def sc_fn(x, ew, src, dst, out0, out1,
              srcva, srcvb, dstva, dstvb, mva, mvb, xva, xvb, acc, *sems):
        s = lax.axis_index("s")
        c = lax.axis_index("c")
        r0 = s * RPW
        ebase = (c * NS + s) * EPW
        bufA = (srcva, dstva, mva, xva) + tuple(sems[:5])
        bufB = (srcvb, dstvb, mvb, xvb) + tuple(sems[5:])

        pltpu.sync_copy(x.at[pl.ds(r0, RPW)], acc.at[pl.ds(r0, RPW)])

        @pl.when(s == NS - 1)
        def _():
            pltpu.sync_copy(x.at[pl.ds(NS * RPW, TAIL)],
                            acc.at[pl.ds(NS * RPW, TAIL)])

        plsc.subcore_barrier()
        _edge_pipeline(x, ew, src, dst, acc, (bufA, bufB), ebase, NCH, K, D)
        plsc.subcore_barrier()

        def writeout(out_ref):
            pltpu.sync_copy(acc.at[pl.ds(r0, RPW)], out_ref.at[pl.ds(r0, RPW)])

            @pl.when(s == NS - 1)
            def _():
                pltpu.sync_copy(acc.at[pl.ds(NS * RPW, TAIL)],
                                out_ref.at[pl.ds(NS * RPW, TAIL)])

        @pl.when(c == 0)
        def _():
            writeout(out0)

        @pl.when(c == 1)
        def _():
            writeout(out1)

    return sc_fn


def _edge_weights_layer(edge_attr, We, be, split):
    E, DE = edge_attr.shape
    D = We.shape[1]
    BE = 2000
    grid = E // BE

    def body(ea_ref, w_ref, b_ref, *o_refs):
        t = jnp.dot(ea_ref[...], w_ref[...],
                    preferred_element_type=jnp.float32) + b_ref[...]
        if split:
            o_refs[0][...] = t[:, :D // 2]
            o_refs[1][...] = t[:, D // 2:]
        else:
            o_refs[0][...] = t

    nw = D // 2 if split else D
    nout = 2 if split else 1
    return pl.pallas_call(
        body, grid=(grid,),
        in_specs=[
            pl.BlockSpec((BE, DE), lambda i: (i, 0)),
            pl.BlockSpec((DE, D), lambda i: (0, 0)),
            pl.BlockSpec((1, D), lambda i: (0, 0)),
        ],
        out_specs=[pl.BlockSpec((BE, nw), lambda i: (i, 0))] * nout,
        out_shape=[jax.ShapeDtypeStruct((E, nw), jnp.float32)] * nout,
    )(edge_attr, We, be.reshape(1, -1))


def _mm_stats(h_lo, h_hi, W1, b1, xsub=None):
    N, Dc = h_lo.shape
    D, DH = W1.shape
    BN_ = 1000
    grid = N // BN_

    def body(lo_ref, hi_ref, *refs):
        if xsub is None:
            w_ref, b_ref, t_ref, st_ref = refs
            w = w_ref[...]
            t = (jnp.dot(lo_ref[...], w[:Dc, :],
                         preferred_element_type=jnp.float32)
                 + jnp.dot(hi_ref[...], w[Dc:, :],
                           preferred_element_type=jnp.float32)
                 + b_ref[...])
        else:
            x_ref, w_ref, b_ref, t_ref, st_ref = refs
            hp = lo_ref[...] + hi_ref[...] - x_ref[...]
            t = jnp.dot(hp, w_ref[...],
                        preferred_element_type=jnp.float32) + b_ref[...]
        t_ref[...] = t
        st = jnp.concatenate(
            [jnp.sum(t, axis=0, keepdims=True),
             jnp.sum(t * t, axis=0, keepdims=True)], axis=0)

        @pl.when(pl.program_id(0) == 0)
        def _():
            st_ref[...] = st

        @pl.when(pl.program_id(0) > 0)
        def _():
            st_ref[...] += st

    in_specs = [
        pl.BlockSpec((BN_, Dc), lambda i: (i, 0)),
        pl.BlockSpec((BN_, Dc), lambda i: (i, 0)),
    ]
    args = [h_lo, h_hi]
    if xsub is not None:
        in_specs.append(pl.BlockSpec((BN_, D), lambda i: (i, 0)))
        args.append(xsub)
    in_specs += [
        pl.BlockSpec((D, DH), lambda i: (0, 0)),
        pl.BlockSpec((1, DH), lambda i: (0, 0)),
    ]
    return pl.pallas_call(
        body, grid=(grid,),
        in_specs=in_specs,
        out_specs=[
            pl.BlockSpec((BN_, DH), lambda i: (i, 0)),
            pl.BlockSpec((2, DH), lambda i: (0, 0)),
        ],
        out_shape=[
            jax.ShapeDtypeStruct((N, DH), jnp.float32),
            jax.ShapeDtypeStruct((2, DH), jnp.float32),
        ],
    )(*args, W1, b1.reshape(1, -1))


def _bn_mm_pool(t, stats, g, bb, W2, b2, bi3, G, want_halves):
    N, DH = t.shape
    BN_ = 1000
    grid = N // BN_
    gb = jnp.stack([g, bb])
    inv_n = 1.0 / N

    def body(t_ref, st_ref, gb_ref, w_ref, b_ref, bi_ref, *o_refs):
        st = st_ref[...]
        mu = st[0:1, :] * inv_n
        var = st[1:2, :] * inv_n - (st[0:1, :] * inv_n) ** 2
        inv = lax.rsqrt(var + 1e-5)
        hn = jnp.maximum(gb_ref[0:1, :] * (t_ref[...] - mu) * inv
                         + gb_ref[1:2, :], 0.0)
        h = jnp.maximum(jnp.dot(hn, w_ref[...],
                                preferred_element_type=jnp.float32)
                        + b_ref[...], 0.0)
        if want_halves:
            o_refs[0][...] = h[:, :DH // 2]
            o_refs[1][...] = h[:, DH // 2:]
        p_ref = o_refs[-1]
        oh = (lax.broadcasted_iota(jnp.int32, (G, BN_), 0)
              == bi_ref[0]).astype(jnp.float32)
        pp = jnp.dot(oh, h, preferred_element_type=jnp.float32)

        @pl.when(pl.program_id(0) == 0)
        def _():
            p_ref[...] = pp

        @pl.when(pl.program_id(0) > 0)
        def _():
            p_ref[...] += pp

    out_specs = []
    out_shapes = []
    if want_halves:
        out_specs += [pl.BlockSpec((BN_, DH // 2), lambda i: (i, 0))] * 2
        out_shapes += [jax.ShapeDtypeStruct((N, DH // 2), jnp.float32)] * 2
    out_specs.append(pl.BlockSpec((G, DH), lambda i: (0, 0)))
    out_shapes.append(jax.ShapeDtypeStruct((G, DH), jnp.float32))
    return pl.pallas_call(
        body, grid=(grid,),
        in_specs=[
            pl.BlockSpec((BN_, DH), lambda i: (i, 0)),
            pl.BlockSpec((2, DH), lambda i: (0, 0)),
            pl.BlockSpec((2, DH), lambda i: (0, 0)),
            pl.BlockSpec((DH, DH), lambda i: (0, 0)),
            pl.BlockSpec((1, DH), lambda i: (0, 0)),
            pl.BlockSpec((1, 1, BN_), lambda i: (i, 0, 0)),
        ],
        out_specs=out_specs,
        out_shape=out_shapes,
    )(t, stats, gb, W2, b2.reshape(1, -1), bi3)


def _head_mm(xin, W, b, nblk, act):
    M, Kd = xin.shape
    _, Nd = W.shape
    BNN = Nd // nblk

    def body(x_ref, w_ref, b_ref, o_ref):
        v = jnp.dot(x_ref[...], w_ref[...],
                    preferred_element_type=jnp.float32) + b_ref[...]
        if act == "relu":
            v = jnp.maximum(v, 0.0)
        else:
            v = jnp.maximum(v, 0.0) + jnp.log(1.0 + jnp.exp(-jnp.abs(v)))
        o_ref[...] = v

    return pl.pallas_call(
        body, grid=(nblk,),
        in_specs=[
            pl.BlockSpec((M, Kd), lambda j: (0, 0)),
            pl.BlockSpec((Kd, BNN), lambda j: (0, j)),
            pl.BlockSpec((1, BNN), lambda j: (0, j)),
        ],
        out_specs=pl.BlockSpec((M, BNN), lambda j: (0, j)),
        out_shape=jax.ShapeDtypeStruct((M, Nd), jnp.float32),
    )(xin, W, b.reshape(1, -1))


def kernel(x, graph_level_feats, edge_attr, edge_index, batch_index, convs, lins):
    N, DF = x.shape
    E = edge_attr.shape[0]
    G = graph_level_feats.shape[0]
    DH = convs[0][4].shape[0]
    src = edge_index[0]
    dst = edge_index[1]
    Ds = [DF, DH, DH, DH]

    ews = [_edge_weights_layer(edge_attr, convs[l][0], convs[l][1], l > 0)
           for l in range(4)]
    bi3 = batch_index.reshape(N // 1000, 1, 1000)

    h_lo, h_hi = None, None
    ps = []
    for l in range(4):
        We, be, W1, b1, g, bb, W2, b2 = convs[l]
        if l == 0:
            sc = _sc_message_pass_esplit(DF, N, E)
            a0, a1 = sc(x, ews[0][0], src, dst)
            t, stats = _mm_stats(a0, a1, W1, b1, xsub=x)
        else:
            sc = _sc_message_pass(Ds[l] // 2, N, E)
            hp_lo, hp_hi = sc(h_lo, h_hi, ews[l][0], ews[l][1], src, dst)
            t, stats = _mm_stats(hp_lo, hp_hi, W1, b1)
        res = _bn_mm_pool(t, stats, g, bb, W2, b2, bi3, G,
                          want_halves=(l < 3))
        if l < 3:
            h_lo, h_hi, p = res
        else:
            (p,) = res
        ps.append(p)

    skip = jnp.reshape(graph_level_feats, (G, -1)).astype(jnp.float32)
    cat = jnp.concatenate(ps + [skip], axis=1)
    Wa, ba, Wc, bc, Wb, bb2 = lins
    z = _head_mm(cat, Wa, ba, 16, "relu")
    z = _head_mm(z, Wc, bc, 16, "relu")
    return _head_mm(z, Wb, bb2, 1, "softplus")


# speedup vs baseline: 3.6097x; 1.8162x over previous
import functools

import jax
import jax.numpy as jnp
from jax import lax
from jax.experimental import pallas as pl
from jax.experimental.pallas import tpu as pltpu
from jax.experimental.pallas import tpu_sc as plsc


def _pick_chunk(epw):
    k = 128 - 128 % 8
    while k >= 8:
        if epw % k == 0 and k % 8 == 0 and (epw // k) % 2 == 1:
            return k
        k -= 8
    raise ValueError(epw)


def _edge_pipeline(x_ref, ew_ref, src, dst, acc, bufs, ebase, NCH, K, Dc):
    def load(cbase, b, first):
        srcv, dstv, mv, xv, s_src, s_dst, s_ew, s_g, s_sc = b
        if not first:
            pltpu.make_async_copy(mv, acc.at[dstv], s_sc).wait()
        pltpu.async_copy(src.at[pl.ds(cbase, K)], srcv, s_src)
        pltpu.async_copy(dst.at[pl.ds(cbase, K)], dstv, s_dst)
        pltpu.async_copy(ew_ref.at[pl.ds(cbase, K)], mv, s_ew)
        pltpu.make_async_copy(src.at[pl.ds(cbase, K)], srcv, s_src).wait()
        pltpu.async_copy(x_ref.at[srcv], xv, s_g)

    def compute(b):
        srcv, dstv, mv, xv, s_src, s_dst, s_ew, s_g, s_sc = b
        pltpu.make_async_copy(ew_ref.at[pl.ds(0, K)], mv, s_ew).wait()
        pltpu.make_async_copy(x_ref.at[srcv], xv, s_g).wait()
        pltpu.make_async_copy(dst.at[pl.ds(0, K)], dstv, s_dst).wait()

        @plsc.parallel_loop(0, K, unroll=4)
        def _row(k):
            for gi in range(Dc // 16):
                sl = pl.ds(gi * 16, 16)
                mv[k, sl] = jnp.maximum(mv[k, sl] + xv[k, sl], 0.0)

        pltpu.async_copy(mv, acc.at[dstv], s_sc, add=True)

    A, B = bufs
    load(ebase, A, True)
    load(ebase + K, B, True)

    def body(i, carry):
        c0 = ebase + 2 * i * K
        compute(A)
        load(c0 + 2 * K, A, False)
        compute(B)
        load(c0 + 3 * K, B, False)
        return carry

    lax.fori_loop(0, (NCH - 3) // 2, body, 0)
    compute(A)
    load(ebase + (NCH - 1) * K, A, False)
    compute(B)
    compute(A)
    pltpu.make_async_copy(A[2], acc.at[A[1]], A[8]).wait()
    pltpu.make_async_copy(B[2], acc.at[B[1]], B[8]).wait()


def _sc_message_pass(Dc, N, E):
    NS = 16
    EPW = E // NS
    K = _pick_chunk(EPW)
    NCH = EPW // K
    RPW = (N // NS) // 8 * 8
    TAIL = N - NS * RPW
    mesh = plsc.VectorSubcoreMesh(core_axis_name="c", subcore_axis_name="s")

    @functools.partial(
        pl.kernel,
        mesh=mesh,
        out_type=[jax.ShapeDtypeStruct((N, Dc), jnp.float32),
                  jax.ShapeDtypeStruct((N, Dc), jnp.float32)],
        scratch_types=(
            [pltpu.VMEM((K,), jnp.int32)] * 2
            + [pltpu.VMEM((K,), jnp.int32)] * 2
            + [pltpu.VMEM((K, Dc), jnp.float32)] * 4
            + [pltpu.VMEM_SHARED((N, Dc), jnp.float32)]
            + [pltpu.SemaphoreType.DMA] * 10
        ),
    )
    def sc_fn(x_lo, x_hi, ew_lo, ew_hi, src, dst, out_lo, out_hi,
              srcva, srcvb, dstva, dstvb, mva, mvb, xva, xvb, acc, *sems):
        s = lax.axis_index("s")
        c = lax.axis_index("c")
        r0 = s * RPW
        ebase = s * EPW
        bufA = (srcva, dstva, mva, xva) + tuple(sems[:5])
        bufB = (srcvb, dstvb, mvb, xvb) + tuple(sems[5:])

        def run(x_ref, ew_ref, out_ref):
            pltpu.sync_copy(x_ref.at[pl.ds(r0, RPW)], acc.at[pl.ds(r0, RPW)])

            @pl.when(s == NS - 1)
            def _():
                pltpu.sync_copy(x_ref.at[pl.ds(NS * RPW, TAIL)],
                                acc.at[pl.ds(NS * RPW, TAIL)])

            plsc.subcore_barrier()
            _edge_pipeline(x_ref, ew_ref, src, dst, acc, (bufA, bufB),
                           ebase, NCH, K, Dc)
            plsc.subcore_barrier()
            pltpu.sync_copy(acc.at[pl.ds(r0, RPW)], out_ref.at[pl.ds(r0, RPW)])

            @pl.when(s == NS - 1)
            def _():
                pltpu.sync_copy(acc.at[pl.ds(NS * RPW, TAIL)],
                                out_ref.at[pl.ds(NS * RPW, TAIL)])

        @pl.when(c == 0)
        def _():
            run(x_lo, ew_lo, out_lo)

        @pl.when(c == 1)
        def _():
            run(x_hi, ew_hi, out_hi)

    return sc_fn


def _sc_message_pass_esplit(D, N, E):
    NS = 16
    NW = 2 * NS
    EPW = E // NW
    K = _pick_chunk(EPW)
    NCH = EPW // K
    RPW = (N // NS) // 8 * 8
    TAIL = N - NS * RPW
    mesh = plsc.VectorSubcoreMesh(core_axis_name="c", subcore_axis_name="s")

    @functools.partial(
        pl.kernel,
        mesh=mesh,
        out_type=[jax.ShapeDtypeStruct((N, D), jnp.float32),
                  jax.ShapeDtypeStruct((N, D), jnp.float32)],
        scratch_types=(
            [pltpu.VMEM((K,), jnp.int32)] * 2
            + [pltpu.VMEM((K,), jnp.int32)] * 2
            + [pltpu.VMEM((K, D), jnp.float32)] * 4
            + [pltpu.VMEM_SHARED((N, D), jnp.float32)]
            + [pltpu.SemaphoreType.DMA] * 10
        ),
    )
    def sc_fn(x, ew, src, dst, out0, out1,
              srcva, srcvb, dstva, dstvb, mva, mvb, xva, xvb, acc, *sems):
        s = lax.axis_index("s")
        c = lax.axis_index("c")
        r0 = s * RPW
        ebase = (c * NS + s) * EPW
        bufA = (srcva, dstva, mva, xva) + tuple(sems[:5])
        bufB = (srcvb, dstvb, mvb, xvb) + tuple(sems[5:])

        pltpu.sync_copy(x.at[pl.ds(r0, RPW)], acc.at[pl.ds(r0, RPW)])

        @pl.when(s == NS - 1)
        def _():
            pltpu.sync_copy(x.at[pl.ds(NS * RPW, TAIL)],
                            acc.at[pl.ds(NS * RPW, TAIL)])

        plsc.subcore_barrier()
        _edge_pipeline(x, ew, src, dst, acc, (bufA, bufB), ebase, NCH, K, D)
        plsc.subcore_barrier()

        def writeout(out_ref):
            pltpu.sync_copy(acc.at[pl.ds(r0, RPW)], out_ref.at[pl.ds(r0, RPW)])

            @pl.when(s == NS - 1)
            def _():
                pltpu.sync_copy(acc.at[pl.ds(NS * RPW, TAIL)],
                                out_ref.at[pl.ds(NS * RPW, TAIL)])

        @pl.when(c == 0)
        def _():
            writeout(out0)

        @pl.when(c == 1)
        def _():
            writeout(out1)

    return sc_fn


def _edge_weights_layer(edge_attr, We, be, split):
    E, DE = edge_attr.shape
    D = We.shape[1]
    BE = 2000
    grid = E // BE

    def body(ea_ref, w_ref, b_ref, *o_refs):
        t = jnp.dot(ea_ref[...], w_ref[...],
                    preferred_element_type=jnp.float32) + b_ref[...]
        if split:
            o_refs[0][...] = t[:, :D // 2]
            o_refs[1][...] = t[:, D // 2:]
        else:
            o_refs[0][...] = t

    nw = D // 2 if split else D
    nout = 2 if split else 1
    return pl.pallas_call(
        body, grid=(grid,),
        in_specs=[
            pl.BlockSpec((BE, DE), lambda i: (i, 0)),
            pl.BlockSpec((DE, D), lambda i: (0, 0)),
            pl.BlockSpec((1, D), lambda i: (0, 0)),
        ],
        out_specs=[pl.BlockSpec((BE, nw), lambda i: (i, 0))] * nout,
        out_shape=[jax.ShapeDtypeStruct((E, nw), jnp.float32)] * nout,
    )(edge_attr, We, be.reshape(1, -1))


def _mm_stats(h_lo, h_hi, W1, b1, xsub=None):
    N, Dc = h_lo.shape
    D, DH = W1.shape
    BN_ = 1000
    grid = N // BN_

    def body(lo_ref, hi_ref, *refs):
        if xsub is None:
            w_ref, b_ref, t_ref, st_ref = refs
            w = w_ref[...]
            t = (jnp.dot(lo_ref[...], w[:Dc, :],
                         preferred_element_type=jnp.float32)
                 + jnp.dot(hi_ref[...], w[Dc:, :],
                           preferred_element_type=jnp.float32)
                 + b_ref[...])
        else:
            x_ref, w_ref, b_ref, t_ref, st_ref = refs
            hp = lo_ref[...] + hi_ref[...] - x_ref[...]
            t = jnp.dot(hp, w_ref[...],
                        preferred_element_type=jnp.float32) + b_ref[...]
        t_ref[...] = t
        st = jnp.concatenate(
            [jnp.sum(t, axis=0, keepdims=True),
             jnp.sum(t * t, axis=0, keepdims=True)], axis=0)

        @pl.when(pl.program_id(0) == 0)
        def _():
            st_ref[...] = st

        @pl.when(pl.program_id(0) > 0)
        def _():
            st_ref[...] += st

    in_specs = [
        pl.BlockSpec((BN_, Dc), lambda i: (i, 0)),
        pl.BlockSpec((BN_, Dc), lambda i: (i, 0)),
    ]
    args = [h_lo, h_hi]
    if xsub is not None:
        in_specs.append(pl.BlockSpec((BN_, D), lambda i: (i, 0)))
        args.append(xsub)
    in_specs += [
        pl.BlockSpec((D, DH), lambda i: (0, 0)),
        pl.BlockSpec((1, DH), lambda i: (0, 0)),
    ]
    return pl.pallas_call(
        body, grid=(grid,),
        in_specs=in_specs,
        out_specs=[
            pl.BlockSpec((BN_, DH), lambda i: (i, 0)),
            pl.BlockSpec((2, DH), lambda i: (0, 0)),
        ],
        out_shape=[
            jax.ShapeDtypeStruct((N, DH), jnp.float32),
            jax.ShapeDtypeStruct((2, DH), jnp.float32),
        ],
    )(*args, W1, b1.reshape(1, -1))


def _bn_mm_pool(t, stats, g, bb, W2, b2, bi3, G, want_halves):
    N, DH = t.shape
    BN_ = 1000
    grid = N // BN_
    gb = jnp.stack([g, bb])
    inv_n = 1.0 / N

    def body(t_ref, st_ref, gb_ref, w_ref, b_ref, bi_ref, *o_refs):
        st = st_ref[...]
        mu = st[0:1, :] * inv_n
        var = st[1:2, :] * inv_n - (st[0:1, :] * inv_n) ** 2
        inv = lax.rsqrt(var + 1e-5)
        hn = jnp.maximum(gb_ref[0:1, :] * (t_ref[...] - mu) * inv
                         + gb_ref[1:2, :], 0.0)
        h = jnp.maximum(jnp.dot(hn, w_ref[...],
                                preferred_element_type=jnp.float32)
                        + b_ref[...], 0.0)
        if want_halves:
            o_refs[0][...] = h[:, :DH // 2]
            o_refs[1][...] = h[:, DH // 2:]
        p_ref = o_refs[-1]
        oh = (lax.broadcasted_iota(jnp.int32, (G, BN_), 0)
              == bi_ref[0]).astype(jnp.float32)
        pp = jnp.dot(oh, h, preferred_element_type=jnp.float32)

        @pl.when(pl.program_id(0) == 0)
        def _():
            p_ref[...] = pp

        @pl.when(pl.program_id(0) > 0)
        def _():
            p_ref[...] += pp

    out_specs = []
    out_shapes = []
    if want_halves:
        out_specs += [pl.BlockSpec((BN_, DH // 2), lambda i: (i, 0))] * 2
        out_shapes += [jax.ShapeDtypeStruct((N, DH // 2), jnp.float32)] * 2
    out_specs.append(pl.BlockSpec((G, DH), lambda i: (0, 0)))
    out_shapes.append(jax.ShapeDtypeStruct((G, DH), jnp.float32))
    return pl.pallas_call(
        body, grid=(grid,),
        in_specs=[
            pl.BlockSpec((BN_, DH), lambda i: (i, 0)),
            pl.BlockSpec((2, DH), lambda i: (0, 0)),
            pl.BlockSpec((2, DH), lambda i: (0, 0)),
            pl.BlockSpec((DH, DH), lambda i: (0, 0)),
            pl.BlockSpec((1, DH), lambda i: (0, 0)),
            pl.BlockSpec((1, 1, BN_), lambda i: (i, 0, 0)),
        ],
        out_specs=out_specs,
        out_shape=out_shapes,
    )(t, stats, gb, W2, b2.reshape(1, -1), bi3)


def _head_mm(xin, W, b, nblk, act):
    M, Kd = xin.shape
    _, Nd = W.shape
    BNN = Nd // nblk

    def body(x_ref, w_ref, b_ref, o_ref):
        v = jnp.dot(x_ref[...], w_ref[...],
                    preferred_element_type=jnp.float32) + b_ref[...]
        if act == "relu":
            v = jnp.maximum(v, 0.0)
        else:
            v = jnp.maximum(v, 0.0) + jnp.log(1.0 + jnp.exp(-jnp.abs(v)))
        o_ref[...] = v

    return pl.pallas_call(
        body, grid=(nblk,),
        in_specs=[
            pl.BlockSpec((M, Kd), lambda j: (0, 0)),
            pl.BlockSpec((Kd, BNN), lambda j: (0, j)),
            pl.BlockSpec((1, BNN), lambda j: (0, j)),
        ],
        out_specs=pl.BlockSpec((M, BNN), lambda j: (0, j)),
        out_shape=jax.ShapeDtypeStruct((M, Nd), jnp.float32),
    )(xin, W, b.reshape(1, -1))


def kernel(x, graph_level_feats, edge_attr, edge_index, batch_index, convs, lins):
    N, DF = x.shape
    E = edge_attr.shape[0]
    G = graph_level_feats.shape[0]
    DH = convs[0][4].shape[0]
    src = edge_index[0]
    dst = edge_index[1]
    Ds = [DF, DH, DH, DH]

    ews = [_edge_weights_layer(edge_attr, convs[l][0], convs[l][1], l > 0)
           for l in range(4)]
    bi3 = batch_index.reshape(N // 1000, 1, 1000)

    h_lo, h_hi = None, None
    ps = []
    for l in range(4):
        We, be, W1, b1, g, bb, W2, b2 = convs[l]
        if l == 0:
            sc = _sc_message_pass_esplit(DF, N, E)
            a0, a1 = sc(x, ews[0][0], src, dst)
            t, stats = _mm_stats(a0, a1, W1, b1, xsub=x)
        else:
            sc = _sc_message_pass(Ds[l] // 2, N, E)
            hp_lo, hp_hi = sc(h_lo, h_hi, ews[l][0], ews[l][1], src, dst)
            t, stats = _mm_stats(hp_lo, hp_hi, W1, b1)
        res = _bn_mm_pool(t, stats, g, bb, W2, b2, bi3, G,
                          want_halves=(l < 3))
        if l < 3:
            h_lo, h_hi, p = res
        else:
            (p,) = res
        ps.append(p)

    skip = jnp.reshape(graph_level_feats, (G, -1)).astype(jnp.float32)
    cat = jnp.concatenate(ps + [skip], axis=1)
    Wa, ba, Wc, bc, Wb, bb2 = lins
    z = _head_mm(cat, Wa, ba, 16, "relu")
    z = _head_mm(z, Wc, bc, 16, "relu")
    return _head_mm(z, Wb, bb2, 1, "softplus")
